# Initial kernel scaffold; baseline (speedup 1.0000x reference)
import jax
import jax.numpy as jnp
from jax.experimental import pallas as pl


def kernel(h, edge_index):
    raise NotImplementedError("write your pallas kernel here")


# speedup vs baseline: 3.3330x; 3.3330x over previous
import functools

import jax
import jax.numpy as jnp
from jax import lax
from jax.experimental import pallas as pl
from jax.experimental.pallas import tpu as pltpu
from jax.experimental.pallas import tpu_sc as plsc

_NUM_NODES = 10000
_DIM = 128
_NUM_EDGES = 320000
_NW = 32
_EPW = _NUM_EDGES // _NW
_CHUNK = 80
_NCHUNK = _EPW // _CHUNK
_GROUPS = _CHUNK // 16
_KB = _DIM // 16


def _decoder_body(h_hbm, src_hbm, dst_hbm, out_hbm,
                  idx_s_v, idx_d_v, rows_s_v, rows_d_v, part_v, out_v, sem):
    wid = lax.axis_index("s") * 2 + lax.axis_index("c")
    wbase = wid * _EPW
    lane = lax.iota(jnp.int32, 16)

    def chunk_body(c, _):
        base = wbase + c * _CHUNK
        pltpu.sync_copy(src_hbm.at[pl.ds(base, _CHUNK)], idx_s_v)
        pltpu.sync_copy(dst_hbm.at[pl.ds(base, _CHUNK)], idx_d_v)
        cp_s = pltpu.async_copy(h_hbm.at[idx_s_v], rows_s_v, sem)
        cp_d = pltpu.async_copy(h_hbm.at[idx_d_v], rows_d_v, sem)
        cp_s.wait()
        cp_d.wait()

        def group_body(g, _):
            for e in range(16):
                row = g * 16 + e
                acc = rows_s_v[row, pl.ds(0, 16)] * rows_d_v[row, pl.ds(0, 16)]
                for k in range(1, _KB):
                    acc = acc + (rows_s_v[row, pl.ds(k * 16, 16)]
                                 * rows_d_v[row, pl.ds(k * 16, 16)])
                part_v[pl.ds(e * 16, 16)] = acc
            lane16 = lane * 16
            tot = plsc.load_gather(part_v, [lane16])
            for k in range(1, 16):
                tot = tot + plsc.load_gather(part_v, [lane16 + k])
            out_v[pl.ds(g * 16, 16)] = 1.0 / (1.0 + jnp.exp(-tot))
            return 0

        lax.fori_loop(0, _GROUPS, group_body, 0)
        pltpu.sync_copy(out_v, out_hbm.at[pl.ds(base, _CHUNK)])
        return 0

    lax.fori_loop(0, _NCHUNK, chunk_body, 0)


@functools.partial(jax.jit, static_argnames=())
def _decode(h, src, dst):
    mesh = plsc.VectorSubcoreMesh(core_axis_name="c", subcore_axis_name="s",
                                  num_cores=2, num_subcores=16)
    f = pl.kernel(
        _decoder_body,
        out_type=jax.ShapeDtypeStruct((_NUM_EDGES,), jnp.float32),
        mesh=mesh,
        scratch_types=[
            pltpu.VMEM((_CHUNK,), jnp.int32),
            pltpu.VMEM((_CHUNK,), jnp.int32),
            pltpu.VMEM((_CHUNK, _DIM), jnp.float32),
            pltpu.VMEM((_CHUNK, _DIM), jnp.float32),
            pltpu.VMEM((256,), jnp.float32),
            pltpu.VMEM((_CHUNK,), jnp.float32),
            pltpu.SemaphoreType.DMA,
        ],
        compiler_params=pltpu.CompilerParams(needs_layout_passes=False),
    )
    return f(h, src, dst)


def kernel(h, edge_index):
    ei = edge_index.astype(jnp.int32)
    return _decode(h, ei[0], ei[1])
